# baseline (speedup 1.0000x reference)
import functools

import jax
import jax.numpy as jnp
from jax import lax
from jax.experimental import pallas as pl
from jax.experimental.pallas import tpu as pltpu
from jax.experimental.pallas import tpu_sc as plsc

VOCAB = 100000
EMBED_DIM = 128
B_ROWS = 4096
B_COLS = 50

NC = 2
NS = 16
NW = NC * NS


def _make_kernel(n_rows):
  b = n_rows * B_COLS
  b_per_w = b // NW
  maj_per_w = n_rows // NW
  chunk = 400
  n_chunks = b_per_w // chunk

  mesh = plsc.VectorSubcoreMesh(
      core_axis_name="c", subcore_axis_name="s",
      num_cores=NC, num_subcores=NS)

  @functools.partial(
      pl.kernel,
      out_type=jax.ShapeDtypeStruct((n_rows, B_COLS, EMBED_DIM), jnp.float32),
      mesh=mesh,
      scratch_types=[
          pltpu.VMEM((b_per_w,), jnp.int32),
          pltpu.VMEM((chunk, EMBED_DIM), jnp.float32),
          pltpu.VMEM((chunk, EMBED_DIM), jnp.float32),
          pltpu.SemaphoreType.DMA,
          pltpu.SemaphoreType.DMA,
          pltpu.SemaphoreType.DMA,
          pltpu.SemaphoreType.DMA,
      ],
  )
  def gather_kernel(idx_hbm, table_hbm, out_hbm, idx_v, rows0, rows1,
                    g0, g1, s0, s1):
    wid = lax.axis_index("s") * NC + lax.axis_index("c")
    base = wid * b_per_w
    maj0 = wid * maj_per_w
    pltpu.sync_copy(idx_hbm.at[pl.ds(base, b_per_w)], idx_v)

    def gather(c, buf, sem):
      return pltpu.make_async_copy(
          table_hbm.at[idx_v.at[pl.ds(c * chunk, chunk)]], buf, sem)

    def put_start(c, buf, sem):
      m0 = maj0 + c * (chunk // B_COLS)
      for j in range(chunk // B_COLS):
        pltpu.make_async_copy(
            buf.at[pl.ds(j * B_COLS, B_COLS)], out_hbm.at[m0 + j], sem
        ).start()

    def put_wait(c, buf, sem):
      m0 = maj0 + c * (chunk // B_COLS)
      for j in range(chunk // B_COLS):
        pltpu.make_async_copy(
            buf.at[pl.ds(j * B_COLS, B_COLS)], out_hbm.at[m0 + j], sem
        ).wait()

    gather(0, rows0, g0).start()
    gather(1, rows1, g1).start()

    def body(i, carry):
      c = i * 2
      gather(c, rows0, g0).wait()
      put_start(c, rows0, s0)
      gather(c + 1, rows1, g1).wait()
      put_start(c + 1, rows1, s1)
      put_wait(c, rows0, s0)

      @pl.when(c + 2 < n_chunks)
      def _():
        gather(c + 2, rows0, g0).start()

      put_wait(c + 1, rows1, s1)

      @pl.when(c + 3 < n_chunks)
      def _():
        gather(c + 3, rows1, g1).start()

      return carry

    lax.fori_loop(0, n_chunks // 2, body, 0)

  return gather_kernel


_N_SPLIT = 2
_ROWS_SPLIT = B_ROWS // _N_SPLIT
_GATHER = _make_kernel(_ROWS_SPLIT)


def kernel(x, table):
  idx = x.astype(jnp.int32)
  parts = [
      _GATHER(idx[i * _ROWS_SPLIT:(i + 1) * _ROWS_SPLIT].reshape(-1), table)
      for i in range(_N_SPLIT)
  ]
  return jnp.concatenate(parts, axis=0)
